# baseline (speedup 1.0000x reference)
import functools

import jax
import jax.numpy as jnp
from jax import lax
from jax.experimental import pallas as pl
from jax.experimental.pallas import tpu as pltpu
from jax.experimental.pallas import tpu_sc as plsc

D_MODEL = 512
N_ROWS = 119
_NC, _NS = 2, 16
_NW = _NC * _NS
_CHUNK = 40
_NGAT = 8


def _proj_body(cbfv_ref, w_ref, b_ref, out_ref):
    out_ref[...] = (
        jnp.dot(cbfv_ref[...], w_ref[...], preferred_element_type=jnp.float32)
        + b_ref[...]
    )


def _compute_proj(cbfv, W, b):
    return pl.pallas_call(
        _proj_body,
        out_shape=jax.ShapeDtypeStruct((N_ROWS, D_MODEL), jnp.float32),
    )(cbfv, W, b.reshape(1, D_MODEL))


@functools.cache
def _make_gather(B):
    bw = B // _NW
    nch = bw // _CHUNK
    assert nch % 2 == 0 and nch >= 6
    mesh = plsc.VectorSubcoreMesh(core_axis_name="c", subcore_axis_name="s")

    @functools.partial(
        pl.kernel,
        out_type=jax.ShapeDtypeStruct((B, D_MODEL), jnp.float32),
        mesh=mesh,
        scratch_types=[
            pltpu.VMEM_SHARED((_NS * bw,), jnp.int32),
            pltpu.VMEM((N_ROWS, D_MODEL), jnp.float32),
            [pltpu.VMEM((_CHUNK, D_MODEL), jnp.float32) for _ in range(2)],
            [pltpu.VMEM((_NGAT,), jnp.int32) for _ in range(2)],
            pltpu.SMEM((2, _CHUNK), jnp.int32),
            [pltpu.SemaphoreType.DMA for _ in range(8)],
        ],
    )
    def k(table_hbm, idx_hbm, out_hbm, idx_sh, table_v, bufs, idx_g, idx_s,
          sems):
        souts, sidx, sidxg, sgat = (sems[:2], sems[2:4], sems[4:6], sems[6:])
        sid = lax.axis_index("s")
        wid = sid * _NC + lax.axis_index("c")
        base = wid * bw
        pltpu.sync_copy(table_hbm, table_v)
        pltpu.sync_copy(idx_hbm.at[pl.ds(base, bw)],
                        idx_sh.at[pl.ds(sid * bw, bw)])

        def stage(chunk, b):
            return pltpu.make_async_copy(
                idx_sh.at[pl.ds(sid * bw + chunk * _CHUNK, _CHUNK)],
                idx_s.at[b], sidx[b])

        def stageg(chunk, b):
            return pltpu.make_async_copy(
                idx_sh.at[pl.ds(sid * bw + chunk * _CHUNK, _NGAT)],
                idx_g[b], sidxg[b])

        def gat(chunk, b):
            return pltpu.make_async_copy(
                table_hbm.at[idx_g[b]], bufs[b].at[pl.ds(0, _NGAT)], sgat[b])

        def outc(chunk, b):
            return pltpu.make_async_copy(
                bufs[b], out_hbm.at[pl.ds(base + chunk * _CHUNK, _CHUNK)],
                souts[b])

        def fill(b):
            def row(r, carry):
                v = idx_s[b, r]
                vals = [table_v[v, pl.ds(c * 16, 16)]
                        for c in range(D_MODEL // 16)]
                for c in range(D_MODEL // 16):
                    bufs[b][r, pl.ds(c * 16, 16)] = vals[c]
                return carry

            lax.fori_loop(_NGAT, _CHUNK, row, 0, unroll=2)

        def step(i, b, first, last):
            stage(i, b).wait()
            stageg(i, b).wait()
            if not first:
                outc(i - 2, b).wait()
            gat(i, b).start()
            fill(b)
            gat(i, b).wait()
            outc(i, b).start()
            if not last:
                stage(i + 2, b).start()
                stageg(i + 2, b).start()

        stage(0, 0).start()
        stageg(0, 0).start()
        stage(1, 1).start()
        stageg(1, 1).start()
        step(0, 0, first=True, last=False)
        step(1, 1, first=True, last=False)

        def body(j, carry):
            for b in range(2):
                step(2 * j + b, b, first=False, last=False)
            return carry

        lax.fori_loop(1, nch // 2 - 1, body, 0)

        step(nch - 2, 0, first=False, last=True)
        step(nch - 1, 1, first=False, last=True)
        outc(nch - 2, 0).wait()
        outc(nch - 1, 1).wait()

    return k


def kernel(src, cbfv, W, b):
    proj = _compute_proj(cbfv, W, b)
    batch, seq = src.shape
    out = _make_gather(batch * seq)(proj, src.T.reshape(-1))
    return out.reshape(seq, batch, D_MODEL).transpose(1, 0, 2)


# speedup vs baseline: 1.0864x; 1.0864x over previous
import functools

import jax
import jax.numpy as jnp
from jax import lax
from jax.experimental import pallas as pl
from jax.experimental.pallas import tpu as pltpu
from jax.experimental.pallas import tpu_sc as plsc

D_MODEL = 512
N_ROWS = 119
_NC, _NS = 2, 16
_NW = _NC * _NS
_CHF = 32
_CHG = 16
_FILL = 8192


def _proj_body(cbfv_ref, w_ref, b_ref, out_ref):
    out_ref[...] = (
        jnp.dot(cbfv_ref[...], w_ref[...], preferred_element_type=jnp.float32)
        + b_ref[...]
    )


def _compute_proj(cbfv, W, b):
    return pl.pallas_call(
        _proj_body,
        out_shape=jax.ShapeDtypeStruct((N_ROWS, D_MODEL), jnp.float32),
    )(cbfv, W, b.reshape(1, D_MODEL))


@functools.cache
def _make_gather(B):
    bw = B // _NW
    gr = bw - _FILL
    nchf = _FILL // _CHF
    nchg = gr // _CHG
    assert nchf == 2 * nchg and nchf % 4 == 0 and nchf >= 12
    mesh = plsc.VectorSubcoreMesh(core_axis_name="c", subcore_axis_name="s")

    @functools.partial(
        pl.kernel,
        out_type=jax.ShapeDtypeStruct((B, D_MODEL), jnp.float32),
        mesh=mesh,
        scratch_types=[
            pltpu.VMEM_SHARED((_NS * _FILL,), jnp.int32),
            pltpu.VMEM((N_ROWS, D_MODEL), jnp.float32),
            [pltpu.VMEM((_CHF, D_MODEL), jnp.float32) for _ in range(2)],
            [pltpu.VMEM((_CHG, D_MODEL), jnp.float32) for _ in range(2)],
            pltpu.VMEM((gr,), jnp.int32),
            pltpu.SMEM((2, _CHF), jnp.int32),
            [pltpu.SemaphoreType.DMA for _ in range(8)],
        ],
    )
    def k(table_hbm, idx_hbm, out_hbm, idx_sh, table_v, fbufs, gbufs, idx_gv,
          idx_s, sems):
        sfo, ssm, sgo, sga = sems[:2], sems[2:4], sems[4:6], sems[6:8]
        sid = lax.axis_index("s")
        wid = sid * _NC + lax.axis_index("c")
        base = wid * bw
        pltpu.sync_copy(table_hbm, table_v)
        pltpu.sync_copy(idx_hbm.at[pl.ds(base, _FILL)],
                        idx_sh.at[pl.ds(sid * _FILL, _FILL)])
        pltpu.sync_copy(idx_hbm.at[pl.ds(base + _FILL, gr)], idx_gv)

        def stage(c, b):
            return pltpu.make_async_copy(
                idx_sh.at[pl.ds(sid * _FILL + c * _CHF, _CHF)], idx_s.at[b],
                ssm[b])

        def fout(c, b):
            return pltpu.make_async_copy(
                fbufs[b], out_hbm.at[pl.ds(base + c * _CHF, _CHF)], sfo[b])

        def gat(g, b):
            return pltpu.make_async_copy(
                table_hbm.at[idx_gv.at[pl.ds(g * _CHG, _CHG)]], gbufs[b],
                sga[b])

        def gout(g, b):
            return pltpu.make_async_copy(
                gbufs[b], out_hbm.at[pl.ds(base + _FILL + g * _CHG, _CHG)],
                sgo[b])

        def fill(b):
            def row(r, carry):
                v = idx_s[b, r]
                vals = [table_v[v, pl.ds(c * 16, 16)]
                        for c in range(D_MODEL // 16)]
                for c in range(D_MODEL // 16):
                    fbufs[b][r, pl.ds(c * 16, 16)] = vals[c]
                return carry

            lax.fori_loop(0, _CHF, row, 0, unroll=2)

        def fstep(i, b, first, last):
            stage(i, b).wait()
            if not first:
                fout(i - 2, b).wait()
            fill(b)
            fout(i, b).start()
            if not last:
                stage(i + 2, b).start()

        def gservice(g, bg, first, last):
            gat(g, bg).wait()
            gout(g, bg).start()
            if not last:
                if not first:
                    gout(g - 1, 1 - bg).wait()
                gat(g + 1, 1 - bg).start()

        stage(0, 0).start()
        stage(1, 1).start()
        gat(0, 0).start()

        gservice(0, 0, first=True, last=False)
        fstep(0, 0, first=True, last=False)
        fstep(1, 1, first=True, last=False)
        gservice(1, 1, first=False, last=False)
        fstep(2, 0, first=False, last=False)
        fstep(3, 1, first=False, last=False)

        def body(u, carry):
            i0 = 4 * u
            gservice(2 * u, 0, first=False, last=False)
            fstep(i0, 0, first=False, last=False)
            fstep(i0 + 1, 1, first=False, last=False)
            gservice(2 * u + 1, 1, first=False, last=False)
            fstep(i0 + 2, 0, first=False, last=False)
            fstep(i0 + 3, 1, first=False, last=False)
            return carry

        lax.fori_loop(1, nchf // 4 - 1, body, 0)

        gservice(nchg - 2, 0, first=False, last=False)
        fstep(nchf - 4, 0, first=False, last=False)
        fstep(nchf - 3, 1, first=False, last=False)
        gservice(nchg - 1, 1, first=False, last=True)
        fstep(nchf - 2, 0, first=False, last=True)
        fstep(nchf - 1, 1, first=False, last=True)

        fout(nchf - 2, 0).wait()
        fout(nchf - 1, 1).wait()
        gout(nchg - 2, 0).wait()
        gout(nchg - 1, 1).wait()

    return k


def kernel(src, cbfv, W, b):
    proj = _compute_proj(cbfv, W, b)
    batch, seq = src.shape
    out = _make_gather(batch * seq)(proj, src.T.reshape(-1))
    return out.reshape(seq, batch, D_MODEL).transpose(1, 0, 2)
